# baseline (speedup 1.0000x reference)
import functools

import jax
import jax.numpy as jnp
from jax import lax
from jax.experimental import pallas as pl
from jax.experimental.pallas import tpu as pltpu
from jax.experimental.pallas import tpu_sc as plsc

NUM_CORES = 2
NUM_SUBCORES = 16
NW = NUM_CORES * NUM_SUBCORES
LANES = 16
BATCH = 16384
FACTORS = 64
B_PER_W = BATCH // NW
CHUNK = 128
NCHUNK = B_PER_W // CHUNK

_MESH = plsc.VectorSubcoreMesh(core_axis_name="c", subcore_axis_name="s")
_PARAMS = pltpu.CompilerParams(
    needs_layout_passes=False, use_tc_tiling_on_sc=False)


def _body_u(users_hbm, gu_hbm, gu_out, uidx_v, gu_v, sem):
  wid = lax.axis_index("s") * NUM_CORES + lax.axis_index("c")
  base = wid * B_PER_W

  pltpu.sync_copy(users_hbm.at[pl.ds(wid * NCHUNK, NCHUNK)], uidx_v)
  copies = [
      pltpu.async_copy(gu_hbm.at[uidx_v.at[j]],
                       gu_v.at[pl.ds(j * CHUNK, CHUNK)], sem)
      for j in range(NCHUNK)
  ]
  for c in copies:
    c.wait()
  pltpu.sync_copy(gu_v, gu_out.at[pl.ds(base, B_PER_W)])


def _body_i(items_hbm, gi_hbm, bi_hbm, gamma_u_hbm,
            xui_out, beta_out, gi_out,
            iidx_v, gu_v, gi_v, bv, xui_v, sem, semb):
  wid = lax.axis_index("s") * NUM_CORES + lax.axis_index("c")
  base = wid * B_PER_W

  pltpu.sync_copy(items_hbm.at[pl.ds(wid * NCHUNK, NCHUNK)], iidx_v)

  copies = [
      pltpu.async_copy(gi_hbm.at[iidx_v.at[j]],
                       gi_v.at[pl.ds(j * CHUNK, CHUNK)], sem)
      for j in range(NCHUNK)
  ] + [
      pltpu.async_copy(bi_hbm.at[iidx_v.at[j]],
                       bv.at[pl.ds(j * CHUNK, CHUNK)], semb)
      for j in range(NCHUNK)
  ]
  pltpu.sync_copy(gamma_u_hbm.at[pl.ds(base, B_PER_W)], gu_v)
  for c in copies:
    c.wait()

  lane = lax.iota(jnp.int32, LANES)

  def group(g, _):
    res = jnp.zeros((LANES,), jnp.float32)
    for t in range(LANES):
      r = g * LANES + t
      acc = gu_v[r, pl.ds(0, LANES)] * gi_v[r, pl.ds(0, LANES)]
      for c in range(1, FACTORS // LANES):
        acc += (gu_v[r, pl.ds(c * LANES, LANES)] *
                gi_v[r, pl.ds(c * LANES, LANES)])
      res = jnp.where(lane == t, jnp.sum(acc), res)
    xui_v[pl.ds(g * LANES, LANES)] = res + bv[pl.ds(g * LANES, LANES)]
    return 0

  lax.fori_loop(0, B_PER_W // LANES, group, 0)

  pltpu.sync_copy(gi_v, gi_out.at[pl.ds(base, B_PER_W)])
  pltpu.sync_copy(bv, beta_out.at[pl.ds(base, B_PER_W)])
  pltpu.sync_copy(xui_v, xui_out.at[pl.ds(base, B_PER_W)])


@jax.jit
def _run(users2, items2, Gu, Gi, bi_flat):
  fa = pl.kernel(
      _body_u,
      out_type=jax.ShapeDtypeStruct((BATCH, FACTORS), jnp.float32),
      mesh=_MESH,
      compiler_params=_PARAMS,
      scratch_types=[
          pltpu.VMEM((NCHUNK, CHUNK), jnp.int32),
          pltpu.VMEM((B_PER_W, FACTORS), jnp.float32),
          pltpu.SemaphoreType.DMA,
      ],
  )
  gamma_u = fa(users2, Gu)

  fb = pl.kernel(
      _body_i,
      out_type=(
          jax.ShapeDtypeStruct((BATCH,), jnp.float32),
          jax.ShapeDtypeStruct((BATCH,), jnp.float32),
          jax.ShapeDtypeStruct((BATCH, FACTORS), jnp.float32),
      ),
      mesh=_MESH,
      compiler_params=_PARAMS,
      scratch_types=[
          pltpu.VMEM((NCHUNK, CHUNK), jnp.int32),
          pltpu.VMEM((B_PER_W, FACTORS), jnp.float32),
          pltpu.VMEM((B_PER_W, FACTORS), jnp.float32),
          pltpu.VMEM((B_PER_W,), jnp.float32),
          pltpu.VMEM((B_PER_W,), jnp.float32),
          pltpu.SemaphoreType.DMA,
          pltpu.SemaphoreType.DMA,
      ],
  )
  xui, beta_i, gamma_i = fb(items2, Gi, bi_flat, gamma_u)
  return xui, beta_i, gamma_u, gamma_i


def kernel(users_indices, items_indices, Gu, Gi, Bi):
  users2 = users_indices.astype(jnp.int32).reshape(BATCH // CHUNK, CHUNK)
  items2 = items_indices.astype(jnp.int32).reshape(BATCH // CHUNK, CHUNK)
  bi_flat = Bi.reshape(Bi.shape[0])
  xui, beta_i, gamma_u, gamma_i = _run(users2, items2, Gu, Gi, bi_flat)
  return (xui, beta_i, gamma_u, gamma_i)


# speedup vs baseline: 1.4973x; 1.4973x over previous
import functools

import jax
import jax.numpy as jnp
import numpy as np
from jax import lax
from jax.experimental import pallas as pl
from jax.experimental.pallas import tpu as pltpu
from jax.experimental.pallas import tpu_sc as plsc

NUM_CORES = 2
NUM_SUBCORES = 16
NW = NUM_CORES * NUM_SUBCORES
LANES = 16
BATCH = 16384
FACTORS = 64
B_PER_W = BATCH // NW
NBLK = B_PER_W // LANES
PASS_ROWS = 256
NPASS = B_PER_W // PASS_ROWS
BPP = PASS_ROWS // LANES


def _body(users_hbm, items_hbm, gu_hbm, gi_hbm, bi_hbm,
          xui_out, beta_out, gu_out, gi_out,
          uidx_v, iidx_v, fu, fi, pu, pi, dummy, bv, xui_v, sem, semb):
  wid = lax.axis_index("s") * NUM_CORES + lax.axis_index("c")
  base = wid * B_PER_W

  pltpu.sync_copy(users_hbm.at[pl.ds(wid * NBLK, NBLK)], uidx_v)
  pltpu.sync_copy(items_hbm.at[pl.ds(wid * NBLK, NBLK)], iidx_v)

  bcopies = [
      pltpu.async_copy(bi_hbm.at[iidx_v.at[b]],
                       bv.at[pl.ds(b * LANES, LANES)], semb)
      for b in range(NBLK)
  ]
  for c in bcopies:
    c.wait()

  lane = lax.iota(jnp.int32, LANES)

  for p in range(NPASS):
    def fire(k, _):
      b = p * BPP + k
      uvec = uidx_v[b, pl.ds(0, LANES)]
      ivec = iidx_v[b, pl.ds(0, LANES)]
      for t in range(LANES):
        u = lax.squeeze(lax.slice(uvec, (t,), (t + 1,)), (0,))
        i = lax.squeeze(lax.slice(ivec, (t,), (t + 1,)), (0,))
        r = k * LANES + t
        pltpu.async_copy(gu_hbm.at[pl.ds(u, 1)], fu.at[pl.ds(r, 1)], sem)
        pltpu.async_copy(gi_hbm.at[pl.ds(i, 1)], fi.at[pl.ds(r, 1)], sem)
      return 0

    lax.fori_loop(0, BPP, fire, 0)

    pltpu.make_async_copy(gu_hbm.at[pl.ds(0, PASS_ROWS)], dummy, sem).wait()
    pltpu.make_async_copy(gi_hbm.at[pl.ds(0, PASS_ROWS)], dummy, sem).wait()

    def group(g, _):
      res = jnp.zeros((LANES,), jnp.float32)
      for t in range(LANES):
        r = g * LANES + t
        prow = g * (LANES // 2) + t // 2
        pcol = (t % 2) * FACTORS
        vu = fu[r, pl.ds(0, LANES)]
        vi = fi[r, pl.ds(0, LANES)]
        pu[prow, pl.ds(pcol, LANES)] = vu
        pi[prow, pl.ds(pcol, LANES)] = vi
        acc = vu * vi
        for c in range(1, FACTORS // LANES):
          vu = fu[r, pl.ds(c * LANES, LANES)]
          vi = fi[r, pl.ds(c * LANES, LANES)]
          pu[prow, pl.ds(pcol + c * LANES, LANES)] = vu
          pi[prow, pl.ds(pcol + c * LANES, LANES)] = vi
          acc += vu * vi
        res = jnp.where(lane == t, jnp.sum(acc), res)
      xui_v[pl.ds(p * PASS_ROWS + g * LANES, LANES)] = (
          res + bv[pl.ds(p * PASS_ROWS + g * LANES, LANES)])
      return 0

    lax.fori_loop(0, BPP, group, 0)

    dst = pl.ds(wid * (B_PER_W // 2) + p * (PASS_ROWS // 2), PASS_ROWS // 2)
    pltpu.sync_copy(pu, gu_out.at[dst])
    pltpu.sync_copy(pi, gi_out.at[dst])

  pltpu.sync_copy(bv, beta_out.at[pl.ds(base, B_PER_W)])
  pltpu.sync_copy(xui_v, xui_out.at[pl.ds(base, B_PER_W)])


@jax.jit
def _run(users2, items2, Gu, Gi, bi_flat):
  mesh = plsc.VectorSubcoreMesh(core_axis_name="c", subcore_axis_name="s")
  f = pl.kernel(
      _body,
      out_type=(
          jax.ShapeDtypeStruct((BATCH,), jnp.float32),
          jax.ShapeDtypeStruct((BATCH,), jnp.float32),
          jax.ShapeDtypeStruct((BATCH // 2, 2 * FACTORS), jnp.float32),
          jax.ShapeDtypeStruct((BATCH // 2, 2 * FACTORS), jnp.float32),
      ),
      mesh=mesh,
      compiler_params=pltpu.CompilerParams(needs_layout_passes=False),
      scratch_types=[
          pltpu.VMEM((NBLK, LANES), jnp.int32),
          pltpu.VMEM((NBLK, LANES), jnp.int32),
          pltpu.VMEM((PASS_ROWS, FACTORS), jnp.float32),
          pltpu.VMEM((PASS_ROWS, FACTORS), jnp.float32),
          pltpu.VMEM((PASS_ROWS // 2, 2 * FACTORS), jnp.float32),
          pltpu.VMEM((PASS_ROWS // 2, 2 * FACTORS), jnp.float32),
          pltpu.VMEM((PASS_ROWS, FACTORS), jnp.float32),
          pltpu.VMEM((B_PER_W,), jnp.float32),
          pltpu.VMEM((B_PER_W,), jnp.float32),
          pltpu.SemaphoreType.DMA,
          pltpu.SemaphoreType.DMA,
      ],
  )
  return f(users2, items2, Gu, Gi, bi_flat)


def kernel(users_indices, items_indices, Gu, Gi, Bi):
  users2 = users_indices.astype(jnp.int32).reshape(BATCH // LANES, LANES)
  items2 = items_indices.astype(jnp.int32).reshape(BATCH // LANES, LANES)
  bi_flat = Bi.reshape(Bi.shape[0])
  xui, beta_i, gu2, gi2 = _run(users2, items2, Gu, Gi, bi_flat)
  gamma_u = gu2.reshape(BATCH, FACTORS)
  gamma_i = gi2.reshape(BATCH, FACTORS)
  return (xui, beta_i, gamma_u, gamma_i)
